# Initial kernel scaffold; baseline (speedup 1.0000x reference)
import jax
import jax.numpy as jnp
from jax.experimental import pallas as pl


def kernel(graph_x, graph_edge, W1a, b1a, W1b, b1b, W2a, b2a, W2b, b2b, Wl, bl, Wp, bp):
    raise NotImplementedError("write your pallas kernel here")


# speedup vs baseline: 5.2335x; 5.2335x over previous
import functools

import jax
import jax.numpy as jnp
from jax import lax
from jax.experimental import pallas as pl
from jax.experimental.pallas import tpu as pltpu
from jax.experimental.pallas import tpu_sc as plsc

_N = 10000
_E = 320000
_F = 64

_NC = 2
_NS = 16
_NW = _NC * _NS
_EPW = _E // _NW
_CH = 80
_NCHUNK = _EPW // _CH
_NP = 10240
_RPT = _NP // _NS


def _segsum_sc(y, src, dst):
    mesh = plsc.VectorSubcoreMesh(core_axis_name="c", subcore_axis_name="s")

    @functools.partial(
        pl.kernel,
        mesh=mesh,
        out_type=jax.ShapeDtypeStruct((_NC, _NP, _F), jnp.float32),
        compiler_params=pltpu.CompilerParams(use_tc_tiling_on_sc=False),
        scratch_types=[
            pltpu.VMEM_SHARED((_NP, _F), jnp.float32),
            pltpu.VMEM((_CH,), jnp.int32),
            pltpu.VMEM((_CH,), jnp.int32),
            pltpu.VMEM((_CH, _F), jnp.float32),
            pltpu.SemaphoreType.DMA,
        ],
    )
    def k(y_hbm, src_hbm, dst_hbm, out_hbm, acc, src_v, dst_v, rows_v, sem):
        c = lax.axis_index("c")
        s = lax.axis_index("s")
        wid = s * _NC + c
        row0 = s * _RPT

        def zrow(r, carry):
            for j in range(_F // 16):
                rows_v[r, pl.ds(j * 16, 16)] = jnp.zeros((16,), jnp.float32)
            return carry
        lax.fori_loop(0, _CH, zrow, 0)
        for j in range(_RPT // _CH):
            pltpu.sync_copy(rows_v, acc.at[pl.ds(row0 + j * _CH, _CH)])
        plsc.subcore_barrier()

        def body(i, carry):
            base = wid * _EPW + i * _CH
            pltpu.sync_copy(src_hbm.at[pl.ds(base, _CH)], src_v)
            pltpu.sync_copy(dst_hbm.at[pl.ds(base, _CH)], dst_v)
            pltpu.async_copy(y_hbm.at[src_v], rows_v, sem).wait()
            pltpu.sync_copy(rows_v, acc.at[dst_v], add=True)
            return carry
        lax.fori_loop(0, _NCHUNK, body, 0)

        plsc.subcore_barrier()
        pltpu.sync_copy(acc.at[pl.ds(row0, _RPT)],
                        out_hbm.at[c, pl.ds(row0, _RPT)])

    return k(y, src, dst)


def _mm_a(x_ref, w_ref, o_ref):
    o_ref[...] = jnp.dot(x_ref[...], w_ref[...], preferred_element_type=jnp.float32)


def _stage_b(y_ref, p_ref, b1a_ref, w1b_ref, b1b_ref, w2a_ref, o_ref):
    agg = (p_ref[0] + p_ref[1])[: _N]
    h = jax.nn.relu(y_ref[...] + agg + b1a_ref[...])
    h = jnp.dot(h, w1b_ref[...], preferred_element_type=jnp.float32) + b1b_ref[...]
    g = jax.nn.relu(h)
    o_ref[...] = jnp.dot(g, w2a_ref[...], preferred_element_type=jnp.float32)


def _stage_c(y_ref, p_ref, b2a_ref, w2b_ref, b2b_ref, wl_ref, bl_ref, wpt_ref, bp_ref, o_ref):
    agg = (p_ref[0] + p_ref[1])[: _N]
    t = jax.nn.relu(y_ref[...] + agg + b2a_ref[...])
    h2 = jnp.dot(t, w2b_ref[...], preferred_element_type=jnp.float32) + b2b_ref[...]
    emb = jax.nn.relu(h2)
    e2 = jax.nn.relu(jnp.dot(emb, wl_ref[...], preferred_element_type=jnp.float32) + bl_ref[...])
    z = jnp.sum(e2 * wpt_ref[...], axis=1, keepdims=True) + bp_ref[...]
    o_ref[...] = 1.0 / (1.0 + jnp.exp(-z))


def kernel(graph_x, graph_edge, W1a, b1a, W1b, b1b, W2a, b2a, W2b, b2b, Wl, bl, Wp, bp):
    src = graph_edge[0]
    dst = graph_edge[1]

    y1 = pl.pallas_call(
        _mm_a, out_shape=jax.ShapeDtypeStruct((_N, _F), jnp.float32),
    )(graph_x, W1a)

    p1 = _segsum_sc(y1, src, dst)

    y2 = pl.pallas_call(
        _stage_b, out_shape=jax.ShapeDtypeStruct((_N, _F), jnp.float32),
    )(y1, p1, b1a.reshape(1, -1), W1b, b1b.reshape(1, -1), W2a)

    p2 = _segsum_sc(y2, src, dst)

    out = pl.pallas_call(
        _stage_c, out_shape=jax.ShapeDtypeStruct((_N, 1), jnp.float32),
    )(y2, p2, b2a.reshape(1, -1), W2b, b2b.reshape(1, -1), Wl,
      bl.reshape(1, -1), Wp.reshape(1, -1), bp.reshape(1, 1))
    return out


# speedup vs baseline: 12.8622x; 2.4577x over previous
import functools

import jax
import jax.numpy as jnp
from jax import lax
from jax.experimental import pallas as pl
from jax.experimental.pallas import tpu as pltpu
from jax.experimental.pallas import tpu_sc as plsc

_N = 10000
_E = 320000
_D = 128
_H = 64

_NC = 2
_NS = 16
_NW = _NC * _NS
_EPW = _E // _NW
_CH = 80
_NP = 10240
_RPT = _NP // _NS
_CPW = _EPW // _CH
_NBUF = 5
_GRP = _CPW // _NBUF


def _segsum_sc(y, src2d, dst2d):
    F = y.shape[1]
    mesh = plsc.VectorSubcoreMesh(core_axis_name="c", subcore_axis_name="s")

    @functools.partial(
        pl.kernel,
        mesh=mesh,
        out_type=jax.ShapeDtypeStruct((_NC, _NP, F), jnp.float32),
        compiler_params=pltpu.CompilerParams(use_tc_tiling_on_sc=False),
        scratch_types=[
            pltpu.VMEM_SHARED((_NP, F), jnp.float32),
            pltpu.VMEM((_CPW, _CH), jnp.int32),
            [pltpu.VMEM((_CH,), jnp.int32) for _ in range(_NBUF)],
            [pltpu.VMEM((_CH, F), jnp.float32) for _ in range(_NBUF)],
            [pltpu.SemaphoreType.DMA for _ in range(_NBUF)],
            [pltpu.SemaphoreType.DMA for _ in range(_NBUF)],
        ],
    )
    def k(y_hbm, src_hbm, dst_hbm, out_hbm, acc, src_v, dbuf, bufs, gsem, dsem):
        c = lax.axis_index("c")
        s = lax.axis_index("s")
        wid = s * _NC + c
        row0 = s * _RPT
        crow0 = wid * _CPW

        gidx = pltpu.async_copy(src_hbm.at[pl.ds(crow0, _CPW)], src_v, gsem[0])

        def zrow(r, carry):
            for j in range(F // 16):
                bufs[0][r, pl.ds(j * 16, 16)] = jnp.zeros((16,), jnp.float32)
            return carry
        lax.fori_loop(0, _CH, zrow, 0)
        for j in range(_RPT // _CH):
            pltpu.sync_copy(bufs[0], acc.at[pl.ds(row0 + j * _CH, _CH)])
        gidx.wait()
        plsc.subcore_barrier()

        for b in range(_NBUF - 1):
            pltpu.async_copy(dst_hbm.at[crow0 + b], dbuf[b], dsem[b])
            pltpu.async_copy(y_hbm.at[src_v.at[b]], bufs[b], gsem[b])

        def group(g, carry):
            for b in range(_NBUF):
                cix = g * _NBUF + b
                b_next = (b + _NBUF - 1) % _NBUF
                @pl.when(cix + _NBUF - 1 < _CPW)
                def _():
                    pltpu.async_copy(
                        dst_hbm.at[crow0 + cix + _NBUF - 1], dbuf[b_next], dsem[b_next])
                    pltpu.async_copy(
                        y_hbm.at[src_v.at[cix + _NBUF - 1]], bufs[b_next], gsem[b_next])
                pltpu.make_async_copy(y_hbm.at[src_v.at[cix]], bufs[b], gsem[b]).wait()
                pltpu.make_async_copy(dst_hbm.at[crow0 + cix], dbuf[b], dsem[b]).wait()
                pltpu.sync_copy(bufs[b], acc.at[dbuf[b]], add=True)
            return carry
        lax.fori_loop(0, _GRP, group, 0)

        plsc.subcore_barrier()
        pltpu.sync_copy(acc.at[pl.ds(row0, _RPT)],
                        out_hbm.at[c, pl.ds(row0, _RPT)])

    return k(y, src2d, dst2d)


def _stage_b(x_ref, pa_ref, pb_ref, w1a_ref, b1a_ref, w1b_ref, b1b_ref, o_ref):
    agg = jnp.concatenate(
        [(pa_ref[0] + pa_ref[1])[: _N], (pb_ref[0] + pb_ref[1])[: _N]], axis=1)
    h = x_ref[...] + agg
    h = jax.nn.relu(jnp.dot(h, w1a_ref[...], preferred_element_type=jnp.float32) + b1a_ref[...])
    h = jnp.dot(h, w1b_ref[...], preferred_element_type=jnp.float32) + b1b_ref[...]
    o_ref[...] = jax.nn.relu(h)


def _stage_c(g_ref, p_ref, w2a_ref, b2a_ref, w2b_ref, b2b_ref, wl_ref, bl_ref, wp_ref, bp_ref, o_ref):
    t = g_ref[...] + (p_ref[0] + p_ref[1])[: _N]
    t = jax.nn.relu(jnp.dot(t, w2a_ref[...], preferred_element_type=jnp.float32) + b2a_ref[...])
    h2 = jnp.dot(t, w2b_ref[...], preferred_element_type=jnp.float32) + b2b_ref[...]
    emb = jax.nn.relu(h2)
    e2 = jax.nn.relu(jnp.dot(emb, wl_ref[...], preferred_element_type=jnp.float32) + bl_ref[...])
    z = jnp.dot(e2, wp_ref[...], preferred_element_type=jnp.float32) + bp_ref[...]
    o_ref[...] = 1.0 / (1.0 + jnp.exp(-z))


def kernel(graph_x, graph_edge, W1a, b1a, W1b, b1b, W2a, b2a, W2b, b2b, Wl, bl, Wp, bp):
    src2d = graph_edge[0].reshape(_E // _CH, _CH)
    dst2d = graph_edge[1].reshape(_E // _CH, _CH)

    p1a = _segsum_sc(graph_x[:, : _H], src2d, dst2d)
    p1b = _segsum_sc(graph_x[:, _H:], src2d, dst2d)

    g = pl.pallas_call(
        _stage_b, out_shape=jax.ShapeDtypeStruct((_N, _H), jnp.float32),
    )(graph_x, p1a, p1b, W1a, b1a.reshape(1, -1), W1b, b1b.reshape(1, -1))

    p2 = _segsum_sc(g, src2d, dst2d)

    out = pl.pallas_call(
        _stage_c, out_shape=jax.ShapeDtypeStruct((_N, 1), jnp.float32),
    )(g, p2, W2a, b2a.reshape(1, -1), W2b, b2b.reshape(1, -1), Wl,
      bl.reshape(1, -1), Wp, bp.reshape(1, 1))
    return out


# speedup vs baseline: 14.6242x; 1.1370x over previous
import functools

import jax
import jax.numpy as jnp
from jax import lax
from jax.experimental import pallas as pl
from jax.experimental.pallas import tpu as pltpu
from jax.experimental.pallas import tpu_sc as plsc

_N = 10000
_E = 320000
_D = 128
_H = 64

_NC = 2
_NS = 16
_NW = _NC * _NS
_EPW = _E // _NW
_CH = 80
_NP = 10240
_RPT = _NP // _NS
_CPW = _EPW // _CH


def _segsum_sc(y, src2d, dst2d, nbuf):
    F = y.shape[1]
    grp = (_CPW + nbuf - 1) // nbuf
    mesh = plsc.VectorSubcoreMesh(core_axis_name="c", subcore_axis_name="s")

    @functools.partial(
        pl.kernel,
        mesh=mesh,
        out_type=jax.ShapeDtypeStruct((_NC, _NP, F), jnp.float32),
        compiler_params=pltpu.CompilerParams(use_tc_tiling_on_sc=False),
        scratch_types=[
            pltpu.VMEM_SHARED((_NP, F), jnp.float32),
            pltpu.VMEM((_CPW, _CH), jnp.int32),
            [pltpu.VMEM((_CH,), jnp.int32) for _ in range(nbuf)],
            [pltpu.VMEM((_CH, F), jnp.float32) for _ in range(nbuf)],
            [pltpu.SemaphoreType.DMA for _ in range(nbuf)],
            [pltpu.SemaphoreType.DMA for _ in range(nbuf)],
        ],
    )
    def k(y_hbm, src_hbm, dst_hbm, out_hbm, acc, src_v, dbuf, bufs, gsem, dsem):
        c = lax.axis_index("c")
        s = lax.axis_index("s")
        wid = s * _NC + c
        row0 = s * _RPT
        crow0 = wid * _CPW

        gidx = pltpu.async_copy(src_hbm.at[pl.ds(crow0, _CPW)], src_v, gsem[0])

        def zrow(r, carry):
            for j in range(F // 16):
                bufs[0][r, pl.ds(j * 16, 16)] = jnp.zeros((16,), jnp.float32)
            return carry
        lax.fori_loop(0, _CH, zrow, 0)
        for j in range(_RPT // _CH):
            pltpu.sync_copy(bufs[0], acc.at[pl.ds(row0 + j * _CH, _CH)])
        gidx.wait()
        plsc.subcore_barrier()

        for b in range(nbuf - 1):
            pltpu.async_copy(dst_hbm.at[crow0 + b], dbuf[b], dsem[b])
            pltpu.async_copy(y_hbm.at[src_v.at[b]], bufs[b], gsem[b])

        def group(g, carry):
            for b in range(nbuf):
                cix = g * nbuf + b
                b_next = (b + nbuf - 1) % nbuf
                @pl.when(cix + nbuf - 1 < _CPW)
                def _():
                    pltpu.async_copy(
                        dst_hbm.at[crow0 + cix + nbuf - 1], dbuf[b_next], dsem[b_next])
                    pltpu.async_copy(
                        y_hbm.at[src_v.at[cix + nbuf - 1]], bufs[b_next], gsem[b_next])
                @pl.when(cix < _CPW)
                def _():
                    pltpu.make_async_copy(y_hbm.at[src_v.at[cix]], bufs[b], gsem[b]).wait()
                    pltpu.make_async_copy(dst_hbm.at[crow0 + cix], dbuf[b], dsem[b]).wait()
                    pltpu.sync_copy(bufs[b], acc.at[dbuf[b]], add=True)
            return carry
        lax.fori_loop(0, grp, group, 0)

        plsc.subcore_barrier()
        pltpu.sync_copy(acc.at[pl.ds(row0, _RPT)],
                        out_hbm.at[c, pl.ds(row0, _RPT)])

    return k(y, src2d, dst2d)


def _stage_b(x_ref, p_ref, w1a_ref, b1a_ref, w1b_ref, b1b_ref, o_ref):
    h = x_ref[...] + (p_ref[0] + p_ref[1])[: _N]
    h = jax.nn.relu(jnp.dot(h, w1a_ref[...], preferred_element_type=jnp.float32) + b1a_ref[...])
    h = jnp.dot(h, w1b_ref[...], preferred_element_type=jnp.float32) + b1b_ref[...]
    o_ref[...] = jax.nn.relu(h)


def _stage_c(g_ref, p_ref, w2a_ref, b2a_ref, w2b_ref, b2b_ref, wl_ref, bl_ref, wp_ref, bp_ref, o_ref):
    t = g_ref[...] + (p_ref[0] + p_ref[1])[: _N]
    t = jax.nn.relu(jnp.dot(t, w2a_ref[...], preferred_element_type=jnp.float32) + b2a_ref[...])
    h2 = jnp.dot(t, w2b_ref[...], preferred_element_type=jnp.float32) + b2b_ref[...]
    emb = jax.nn.relu(h2)
    e2 = jax.nn.relu(jnp.dot(emb, wl_ref[...], preferred_element_type=jnp.float32) + bl_ref[...])
    z = jnp.dot(e2, wp_ref[...], preferred_element_type=jnp.float32) + bp_ref[...]
    o_ref[...] = 1.0 / (1.0 + jnp.exp(-z))


def kernel(graph_x, graph_edge, W1a, b1a, W1b, b1b, W2a, b2a, W2b, b2b, Wl, bl, Wp, bp):
    src2d = graph_edge[0].reshape(_E // _CH, _CH)
    dst2d = graph_edge[1].reshape(_E // _CH, _CH)

    p1 = _segsum_sc(graph_x, src2d, dst2d, nbuf=3)

    g = pl.pallas_call(
        _stage_b, out_shape=jax.ShapeDtypeStruct((_N, _H), jnp.float32),
    )(graph_x, p1, W1a, b1a.reshape(1, -1), W1b, b1b.reshape(1, -1))

    p2 = _segsum_sc(g, src2d, dst2d, nbuf=5)

    out = pl.pallas_call(
        _stage_c, out_shape=jax.ShapeDtypeStruct((_N, 1), jnp.float32),
    )(g, p2, W2a, b2a.reshape(1, -1), W2b, b2b.reshape(1, -1), Wl,
      bl.reshape(1, -1), Wp, bp.reshape(1, 1))
    return out
